# baseline (speedup 1.0000x reference)
import functools

import jax
import jax.numpy as jnp
from jax import lax
from jax.experimental import pallas as pl
from jax.experimental.pallas import tpu as pltpu
from jax.experimental.pallas import tpu_sc as plsc

_NUM_CODES = 8192
_DIM = 256
_ROWS = 8192
_TILE = 512
_GRID = _ROWS // _TILE
_NC = 2
_NS = 16
_BPW = _ROWS // (_NC * _NS)
_CHUNK = 128

_LOG2E = 1.4426950408889634


def _dist_body(z_ref, e_ref, p_ref, idx_ref, loss_ref, acc_ref):
    z = z_ref[...]
    e = e_ref[...]
    z2 = jnp.sum(z * z, axis=1, keepdims=True)
    e2 = jnp.sum(e * e, axis=1)
    mm = lax.dot_general(z * -2.0, e, (((1,), (1,)), ((), ())),
                         preferred_element_type=jnp.float32)
    d = (z2 + e2[None, :]) + mm
    dmin = jnp.min(d, axis=1, keepdims=True)
    t = d - dmin
    ids8 = lax.broadcasted_iota(jnp.int32, (8, _NUM_CODES), 1).astype(
        jnp.float32)
    t3 = t.reshape(_TILE // 8, 8, _NUM_CODES)
    idxf = jnp.min(jnp.where(t3 == 0.0, ids8[None], jnp.float32(3e9)), axis=2)
    idx = idxf.reshape(_TILE).astype(jnp.int32)
    un = jnp.exp2(t * (-2.0 * _LOG2E))
    rcp = 1.0 / jnp.sum(un, axis=1, keepdims=True)
    p_ref[...] = un * rcp
    idx_ref[0, 0, :] = idx

    @pl.when(pl.program_id(0) == 0)
    def _():
        acc_ref[0] = 0.0

    acc_ref[0] += jnp.sum(dmin)

    @pl.when(pl.program_id(0) == pl.num_programs(0) - 1)
    def _():
        loss_ref[0, 0] = acc_ref[0]


def _dist_body_aliased(z_ref, e_ref, probin_ref, p_ref, idx_ref, loss_ref,
                       acc_ref):
    del probin_ref
    _dist_body(z_ref, e_ref, p_ref, idx_ref, loss_ref, acc_ref)


def _make_gather(n_rows):
    mesh = plsc.VectorSubcoreMesh(core_axis_name="c", subcore_axis_name="s")
    bpw = n_rows // (_NC * _NS)

    @functools.partial(
        pl.kernel, mesh=mesh,
        out_type=jax.ShapeDtypeStruct((n_rows, _DIM), jnp.float32),
        scratch_types=[
            pltpu.VMEM((_CHUNK,), jnp.int32),
            pltpu.VMEM((_CHUNK,), jnp.int32),
            pltpu.VMEM((_CHUNK, _DIM), jnp.float32),
            pltpu.VMEM((_CHUNK, _DIM), jnp.float32),
            pltpu.SemaphoreType.DMA,
            pltpu.SemaphoreType.DMA,
        ],
    )
    def gather(emb_hbm, idx_hbm, out_hbm, idx_a, idx_b, rows_a, rows_b,
               sem_a, sem_b):
        wid = lax.axis_index("s") * _NC + lax.axis_index("c")
        base = wid * bpw
        bufs = ((idx_a, rows_a, sem_a), (idx_b, rows_b, sem_b))
        cps = []
        for j in range(bpw // _CHUNK):
            off = base + j * _CHUNK
            idx_v, rows_v, sem = bufs[j]
            pltpu.sync_copy(
                idx_hbm.at[off // _TILE, 0, pl.ds(off % _TILE, _CHUNK)],
                idx_v)
            cps.append(pltpu.async_copy(emb_hbm.at[idx_v], rows_v, sem))
        for j in range(bpw // _CHUNK):
            off = base + j * _CHUNK
            cps[j].wait()
            pltpu.sync_copy(bufs[j][1], out_hbm.at[pl.ds(off, _CHUNK)])

    return gather


@functools.cache
def _gather_fn(n_rows):
    return _make_gather(n_rows)


_HGRID = _GRID // 2
_HROWS = _ROWS // 2


def kernel(z, embedding):
    b, c, h, w = z.shape
    zf = jnp.transpose(z, (0, 2, 3, 1)).reshape(-1, c)
    common = dict(
        scratch_shapes=[pltpu.SMEM((1,), jnp.float32)],
        compiler_params=pltpu.CompilerParams(
            dimension_semantics=("arbitrary",)),
    )
    out_specs = [
        pl.BlockSpec((1, 1, _TILE), lambda i: (i, 0, 0)),
        pl.BlockSpec(memory_space=pltpu.SMEM),
    ]
    out_shape = [
        jax.ShapeDtypeStruct((_HGRID, 1, _TILE), jnp.int32),
        jax.ShapeDtypeStruct((1, 1), jnp.float32),
    ]
    prob, idx3_a, lsum_a = pl.pallas_call(
        _dist_body,
        grid=(_HGRID,),
        in_specs=[
            pl.BlockSpec((_TILE, _DIM), lambda i: (i, 0)),
            pl.BlockSpec((_NUM_CODES, _DIM), lambda i: (0, 0)),
        ],
        out_specs=[pl.BlockSpec((_TILE, _NUM_CODES), lambda i: (i, 0)),
                   *out_specs],
        out_shape=[jax.ShapeDtypeStruct((_ROWS, _NUM_CODES), jnp.float32),
                   *out_shape],
        **common,
    )(zf, embedding)
    zq_a = _gather_fn(_HROWS)(embedding, idx3_a)
    prob, idx3_b, lsum_b = pl.pallas_call(
        _dist_body_aliased,
        grid=(_HGRID,),
        in_specs=[
            pl.BlockSpec((_TILE, _DIM), lambda i: (i + _HGRID, 0)),
            pl.BlockSpec((_NUM_CODES, _DIM), lambda i: (0, 0)),
            pl.BlockSpec(memory_space=pl.ANY),
        ],
        out_specs=[pl.BlockSpec((_TILE, _NUM_CODES),
                                lambda i: (i + _HGRID, 0)),
                   *out_specs],
        out_shape=[jax.ShapeDtypeStruct((_ROWS, _NUM_CODES), jnp.float32),
                   *out_shape],
        input_output_aliases={2: 0},
        **common,
    )(zf, embedding, prob)
    zq_b = _gather_fn(_HROWS)(embedding, idx3_b)
    zq = jnp.concatenate([zq_a, zq_b], axis=0)
    zq_out = jnp.transpose(zq.reshape(b, h, w, c), (0, 3, 1, 2))
    q_loss = (lsum_a[0, 0] + lsum_b[0, 0]) * (1.25 / (_ROWS * _DIM))
    return (zq_out, q_loss, prob)


# speedup vs baseline: 1.1221x; 1.1221x over previous
import functools

import jax
import jax.numpy as jnp
from jax import lax
from jax.experimental import pallas as pl
from jax.experimental.pallas import tpu as pltpu
from jax.experimental.pallas import tpu_sc as plsc

_NUM_CODES = 8192
_DIM = 256
_ROWS = 8192
_TILE = 512
_GRID = _ROWS // _TILE
_NC = 2
_NS = 16
_BPW = _ROWS // (_NC * _NS)
_CHUNK = 128

_LOG2E = 1.4426950408889634


def _dist_body(z_ref, e_ref, p_ref, idx_ref, loss_ref, acc_ref):
    z = z_ref[...]
    e = e_ref[...]
    z2 = jnp.sum(z * z, axis=1, keepdims=True)
    e2 = jnp.sum(e * e, axis=1)
    mm = lax.dot_general(z * -2.0, e, (((1,), (1,)), ((), ())),
                         preferred_element_type=jnp.float32)
    d = (z2 + e2[None, :]) + mm
    dmin = jnp.min(d, axis=1, keepdims=True)
    t = d - dmin
    ids8 = lax.broadcasted_iota(jnp.int32, (8, _NUM_CODES), 1).astype(
        jnp.float32)
    t3 = t.reshape(_TILE // 8, 8, _NUM_CODES)
    idxf = jnp.min(jnp.where(t3 == 0.0, ids8[None], jnp.float32(3e9)), axis=2)
    idx = idxf.reshape(_TILE).astype(jnp.int32)
    un = jnp.exp2(t * (-2.0 * _LOG2E))
    rcp = 1.0 / jnp.sum(un, axis=1, keepdims=True)
    p_ref[...] = un * rcp
    idx_ref[0, 0, :] = idx

    @pl.when(pl.program_id(0) == 0)
    def _():
        acc_ref[0] = 0.0

    acc_ref[0] += jnp.sum(dmin)

    @pl.when(pl.program_id(0) == _GRID - 1)
    def _():
        loss_ref[0, 0] = acc_ref[0] * (1.25 / (_ROWS * _DIM))


def _make_gather():
    mesh = plsc.VectorSubcoreMesh(core_axis_name="c", subcore_axis_name="s")

    @functools.partial(
        pl.kernel, mesh=mesh,
        out_type=jax.ShapeDtypeStruct((_ROWS, _DIM), jnp.float32),
        scratch_types=[
            pltpu.VMEM((_CHUNK,), jnp.int32),
            pltpu.VMEM((_CHUNK,), jnp.int32),
            pltpu.VMEM((_CHUNK, _DIM), jnp.float32),
            pltpu.VMEM((_CHUNK, _DIM), jnp.float32),
            pltpu.SemaphoreType.DMA,
            pltpu.SemaphoreType.DMA,
        ],
    )
    def gather(emb_hbm, idx_hbm, out_hbm, idx_a, idx_b, rows_a, rows_b,
               sem_a, sem_b):
        wid = lax.axis_index("s") * _NC + lax.axis_index("c")
        base = wid * _BPW
        off_a, off_b = base, base + _CHUNK
        pltpu.sync_copy(
            idx_hbm.at[off_a // _TILE, 0, pl.ds(off_a % _TILE, _CHUNK)], idx_a)
        cp_a = pltpu.async_copy(emb_hbm.at[idx_a], rows_a, sem_a)
        pltpu.sync_copy(
            idx_hbm.at[off_b // _TILE, 0, pl.ds(off_b % _TILE, _CHUNK)], idx_b)
        cp_b = pltpu.async_copy(emb_hbm.at[idx_b], rows_b, sem_b)
        cp_a.wait()
        pltpu.sync_copy(rows_a, out_hbm.at[pl.ds(off_a, _CHUNK)])
        cp_b.wait()
        pltpu.sync_copy(rows_b, out_hbm.at[pl.ds(off_b, _CHUNK)])

    return gather


@functools.cache
def _gather_fn():
    return _make_gather()


def kernel(z, embedding):
    b, c, h, w = z.shape
    zf = jnp.transpose(z, (0, 2, 3, 1)).reshape(-1, c)
    prob, idx3, loss = pl.pallas_call(
        _dist_body,
        grid=(_GRID,),
        in_specs=[
            pl.BlockSpec((_TILE, _DIM), lambda i: (i, 0)),
            pl.BlockSpec((_NUM_CODES, _DIM), lambda i: (0, 0)),
        ],
        out_specs=[
            pl.BlockSpec((_TILE, _NUM_CODES), lambda i: (i, 0)),
            pl.BlockSpec((1, 1, _TILE), lambda i: (i, 0, 0)),
            pl.BlockSpec(memory_space=pltpu.SMEM),
        ],
        out_shape=[
            jax.ShapeDtypeStruct((_ROWS, _NUM_CODES), jnp.float32),
            jax.ShapeDtypeStruct((_GRID, 1, _TILE), jnp.int32),
            jax.ShapeDtypeStruct((1, 1), jnp.float32),
        ],
        scratch_shapes=[pltpu.SMEM((1,), jnp.float32)],
        compiler_params=pltpu.CompilerParams(
            dimension_semantics=("arbitrary",)),
    )(zf, embedding)
    zq = _gather_fn()(embedding, idx3)
    zq_out = jnp.transpose(zq.reshape(b, h, w, c), (0, 3, 1, 2))
    return (zq_out, loss[0, 0], prob)
